# baseline (speedup 1.0000x reference)
import functools

import jax
import jax.numpy as jnp
from jax import lax
from jax.experimental import pallas as pl
from jax.experimental.pallas import tpu as pltpu
from jax.experimental.pallas import tpu_sc as plsc

H = 1024
EPS = 1e-12

_NC = 2
_NS = 16
_NW = _NC * _NS

_CH = 32
_BLK = 512
_SLABS = ((0, 1), (1, 1), (2, 1), (3, 1))


def _sc_gather_slab(table, flat_ids, S, s_slab, lo, slab_rows):
    b_per_w = slab_rows // _NW
    ch = min(_CH, b_per_w)
    n_ch = b_per_w // ch
    mesh = plsc.VectorSubcoreMesh(core_axis_name="c", subcore_axis_name="s")

    scratch = [pltpu.VMEM((b_per_w,), jnp.int32),
               pltpu.VMEM((ch, H), jnp.float32)]
    if n_ch > 1:
        scratch.append(pltpu.VMEM((ch, H), jnp.float32))
    scratch += [pltpu.SemaphoreType.DMA] * (2 if n_ch == 1 else 4)

    @functools.partial(
        pl.kernel,
        mesh=mesh,
        out_type=jax.ShapeDtypeStruct((slab_rows, H), jnp.float32),
        scratch_types=scratch,
    )
    def gather_kernel(table_hbm, idx_hbm, out_hbm, idx_v, *bufs):
        if n_ch == 1:
            r0, gs0, ws0 = bufs
            row_b, gsem, wsem = (r0,), (gs0,), (ws0,)
        else:
            r0, r1, gs0, gs1, ws0, ws1 = bufs
            row_b, gsem, wsem = (r0, r1), (gs0, gs1), (ws0, ws1)

        wid = lax.axis_index("s") * _NC + lax.axis_index("c")
        r_base = wid * b_per_w
        b = r_base // s_slab
        flat_base = b * S + lo + (r_base % s_slab)
        pltpu.sync_copy(idx_hbm.at[pl.ds(flat_base, b_per_w)], idx_v)

        gathers = [None] * n_ch
        writes = [None] * n_ch
        for c in range(n_ch):
            bb = c % 2
            if c >= 2:
                writes[c - 2].wait()
            gathers[c] = pltpu.async_copy(
                table_hbm.at[idx_v.at[pl.ds(c * ch, ch)]], row_b[bb],
                gsem[bb])
            if c >= 1:
                pb = (c - 1) % 2
                gathers[c - 1].wait()
                writes[c - 1] = pltpu.async_copy(
                    row_b[pb], out_hbm.at[pl.ds(r_base + (c - 1) * ch, ch)],
                    wsem[pb])
        gathers[n_ch - 1].wait()
        writes[n_ch - 1] = pltpu.async_copy(
            row_b[(n_ch - 1) % 2],
            out_hbm.at[pl.ds(r_base + (n_ch - 1) * ch, ch)],
            wsem[(n_ch - 1) % 2])
        if n_ch >= 2:
            writes[n_ch - 2].wait()
        writes[n_ch - 1].wait()

    return gather_kernel(table, flat_ids)


def _ln_math(x_ref, tt_ref, pos_ref, ttab_ref, gamma_ref, beta_ref, o_ref):
    t0 = ttab_ref[0:1, :]
    td = ttab_ref[1:2, :] - t0
    t = tt_ref[:, 0:1]
    x = x_ref[...] + pos_ref[...] + t0 + t * td
    mean = jnp.mean(x, axis=1, keepdims=True)
    xc = x - mean
    var = jnp.mean(xc * xc, axis=1, keepdims=True)
    xn = xc * lax.rsqrt(var + EPS)
    o_ref[...] = xn * gamma_ref[...] + beta_ref[...]


def _ln_body_first(x_ref, tt_ref, pos_ref, ttab_ref, gamma_ref, beta_ref,
                   o_ref):
    _ln_math(x_ref, tt_ref, pos_ref, ttab_ref, gamma_ref, beta_ref, o_ref)


def _ln_body_acc(x_ref, tt_ref, pos_ref, ttab_ref, gamma_ref, beta_ref,
                 acc_ref, o_ref):
    del acc_ref
    _ln_math(x_ref, tt_ref, pos_ref, ttab_ref, gamma_ref, beta_ref, o_ref)


def _tc_ln_slab(base_blk, n_rows, B, s_slab_blocks, s_blocks, g, tt_all, pos,
                ttab, gamma2, beta2, acc):
    in_specs = [
        pl.BlockSpec((_BLK, H), lambda i, j: (j * s_slab_blocks + i, 0)),
        pl.BlockSpec((_BLK, 1), lambda i, j: (j * s_blocks + base_blk + i, 0)),
        pl.BlockSpec((_BLK, H), lambda i, j: (base_blk + i, 0)),
        pl.BlockSpec((2, H), lambda i, j: (0, 0)),
        pl.BlockSpec((1, H), lambda i, j: (0, 0)),
        pl.BlockSpec((1, H), lambda i, j: (0, 0)),
    ]
    out_spec = pl.BlockSpec(
        (_BLK, H), lambda i, j: (j * s_blocks + base_blk + i, 0))
    cp = pltpu.CompilerParams(dimension_semantics=("arbitrary", "arbitrary"))
    args = (g, tt_all, pos, ttab, gamma2, beta2)
    if acc is None:
        return pl.pallas_call(
            _ln_body_first,
            grid=(s_slab_blocks, B),
            in_specs=in_specs,
            out_specs=out_spec,
            out_shape=jax.ShapeDtypeStruct((n_rows, H), jnp.float32),
            compiler_params=cp,
        )(*args)
    return pl.pallas_call(
        _ln_body_acc,
        grid=(s_slab_blocks, B),
        in_specs=in_specs + [pl.BlockSpec(memory_space=pl.ANY)],
        out_specs=out_spec,
        out_shape=jax.ShapeDtypeStruct((n_rows, H), jnp.float32),
        input_output_aliases={6: 0},
        compiler_params=cp,
    )(*args, acc)


def kernel(input_ids, position_ids, token_type_ids, word_embeddings,
           position_embeddings, token_type_embeddings, ln_gamma, ln_beta):
    B, S = input_ids.shape
    n_rows = B * S
    s_blocks = S // _BLK

    flat_ids = input_ids.reshape(n_rows).astype(jnp.int32)
    tt_all = token_type_ids.reshape(n_rows, 1).astype(jnp.float32)
    pos = position_embeddings[:S]
    gamma2 = ln_gamma.reshape(1, H)
    beta2 = ln_beta.reshape(1, H)

    gs = []
    for off, size in _SLABS:
        s_slab = size * _BLK
        gs.append(_sc_gather_slab(word_embeddings, flat_ids, S, s_slab,
                                  off * _BLK, B * s_slab))

    acc = None
    for k, (off, size) in enumerate(_SLABS):
        acc = _tc_ln_slab(off, n_rows, B, size, s_blocks, gs[k], tt_all, pos,
                          token_type_embeddings, gamma2, beta2, acc)

    return acc.reshape(B, S, H)


# speedup vs baseline: 1.0355x; 1.0355x over previous
import functools

import jax
import jax.numpy as jnp
from jax import lax
from jax.experimental import pallas as pl
from jax.experimental.pallas import tpu as pltpu
from jax.experimental.pallas import tpu_sc as plsc

H = 1024
EPS = 1e-12

_NC = 2
_NS = 16
_NW = _NC * _NS

_CH = 32
_BLK = 512
_SLABS = ((0, 2), (2, 2))


def _sc_gather_slab(table, flat_ids, S, s_slab, lo, slab_rows):
    b_per_w = slab_rows // _NW
    ch = min(_CH, b_per_w)
    n_ch = b_per_w // ch
    mesh = plsc.VectorSubcoreMesh(core_axis_name="c", subcore_axis_name="s")

    scratch = [pltpu.VMEM((b_per_w,), jnp.int32),
               pltpu.VMEM((ch, H), jnp.float32)]
    if n_ch > 1:
        scratch.append(pltpu.VMEM((ch, H), jnp.float32))
    scratch += [pltpu.SemaphoreType.DMA] * (2 if n_ch == 1 else 4)

    @functools.partial(
        pl.kernel,
        mesh=mesh,
        out_type=jax.ShapeDtypeStruct((slab_rows, H), jnp.float32),
        scratch_types=scratch,
    )
    def gather_kernel(table_hbm, idx_hbm, out_hbm, idx_v, *bufs):
        if n_ch == 1:
            r0, gs0, ws0 = bufs
            row_b, gsem, wsem = (r0,), (gs0,), (ws0,)
        else:
            r0, r1, gs0, gs1, ws0, ws1 = bufs
            row_b, gsem, wsem = (r0, r1), (gs0, gs1), (ws0, ws1)

        wid = lax.axis_index("s") * _NC + lax.axis_index("c")
        r_base = wid * b_per_w
        b = r_base // s_slab
        flat_base = b * S + lo + (r_base % s_slab)
        pltpu.sync_copy(idx_hbm.at[pl.ds(flat_base, b_per_w)], idx_v)

        gathers = [None] * n_ch
        writes = [None] * n_ch
        for c in range(n_ch):
            bb = c % 2
            if c >= 2:
                writes[c - 2].wait()
            gathers[c] = pltpu.async_copy(
                table_hbm.at[idx_v.at[pl.ds(c * ch, ch)]], row_b[bb],
                gsem[bb])
            if c >= 1:
                pb = (c - 1) % 2
                gathers[c - 1].wait()
                writes[c - 1] = pltpu.async_copy(
                    row_b[pb], out_hbm.at[pl.ds(r_base + (c - 1) * ch, ch)],
                    wsem[pb])
        gathers[n_ch - 1].wait()
        writes[n_ch - 1] = pltpu.async_copy(
            row_b[(n_ch - 1) % 2],
            out_hbm.at[pl.ds(r_base + (n_ch - 1) * ch, ch)],
            wsem[(n_ch - 1) % 2])
        if n_ch >= 2:
            writes[n_ch - 2].wait()
        writes[n_ch - 1].wait()

    return gather_kernel(table, flat_ids)


def _ln_math(x_ref, tt_ref, pos_ref, ttab_ref, gamma_ref, beta_ref, o_ref):
    t0 = ttab_ref[0:1, :]
    td = ttab_ref[1:2, :] - t0
    t = tt_ref[:, 0:1]
    x = x_ref[...] + pos_ref[...] + t0 + t * td
    mean = jnp.mean(x, axis=1, keepdims=True)
    xc = x - mean
    var = jnp.mean(xc * xc, axis=1, keepdims=True)
    xn = xc * lax.rsqrt(var + EPS)
    o_ref[...] = xn * gamma_ref[...] + beta_ref[...]


def _ln_body_first(x_ref, tt_ref, pos_ref, ttab_ref, gamma_ref, beta_ref,
                   o_ref):
    _ln_math(x_ref, tt_ref, pos_ref, ttab_ref, gamma_ref, beta_ref, o_ref)


def _ln_body_acc(x_ref, tt_ref, pos_ref, ttab_ref, gamma_ref, beta_ref,
                 acc_ref, o_ref):
    del acc_ref
    _ln_math(x_ref, tt_ref, pos_ref, ttab_ref, gamma_ref, beta_ref, o_ref)


def _tc_ln_slab(base_blk, n_rows, B, s_slab_blocks, s_blocks, g, tt_all, pos,
                ttab, gamma2, beta2, acc):
    in_specs = [
        pl.BlockSpec((_BLK, H), lambda i, j: (j * s_slab_blocks + i, 0)),
        pl.BlockSpec((_BLK, 1), lambda i, j: (j * s_blocks + base_blk + i, 0)),
        pl.BlockSpec((_BLK, H), lambda i, j: (base_blk + i, 0)),
        pl.BlockSpec((2, H), lambda i, j: (0, 0)),
        pl.BlockSpec((1, H), lambda i, j: (0, 0)),
        pl.BlockSpec((1, H), lambda i, j: (0, 0)),
    ]
    out_spec = pl.BlockSpec(
        (_BLK, H), lambda i, j: (j * s_blocks + base_blk + i, 0))
    cp = pltpu.CompilerParams(dimension_semantics=("arbitrary", "arbitrary"))
    args = (g, tt_all, pos, ttab, gamma2, beta2)
    if acc is None:
        return pl.pallas_call(
            _ln_body_first,
            grid=(s_slab_blocks, B),
            in_specs=in_specs,
            out_specs=out_spec,
            out_shape=jax.ShapeDtypeStruct((n_rows, H), jnp.float32),
            compiler_params=cp,
        )(*args)
    return pl.pallas_call(
        _ln_body_acc,
        grid=(s_slab_blocks, B),
        in_specs=in_specs + [pl.BlockSpec(memory_space=pl.ANY)],
        out_specs=out_spec,
        out_shape=jax.ShapeDtypeStruct((n_rows, H), jnp.float32),
        input_output_aliases={6: 0},
        compiler_params=cp,
    )(*args, acc)


def kernel(input_ids, position_ids, token_type_ids, word_embeddings,
           position_embeddings, token_type_embeddings, ln_gamma, ln_beta):
    B, S = input_ids.shape
    n_rows = B * S
    s_blocks = S // _BLK

    flat_ids = input_ids.reshape(n_rows).astype(jnp.int32)
    tt_all = token_type_ids.reshape(n_rows, 1).astype(jnp.float32)
    pos = position_embeddings[:S]
    gamma2 = ln_gamma.reshape(1, H)
    beta2 = ln_beta.reshape(1, H)

    gs = []
    for off, size in _SLABS:
        s_slab = size * _BLK
        gs.append(_sc_gather_slab(word_embeddings, flat_ids, S, s_slab,
                                  off * _BLK, B * s_slab))

    acc = None
    for k, (off, size) in enumerate(_SLABS):
        acc = _tc_ln_slab(off, n_rows, B, size, s_blocks, gs[k], tt_all, pos,
                          token_type_embeddings, gamma2, beta2, acc)

    return acc.reshape(B, S, H)
